# Initial kernel scaffold; baseline (speedup 1.0000x reference)
import jax
import jax.numpy as jnp
from jax.experimental import pallas as pl


def kernel(x, edge_index, W1, b1, W2, b2):
    raise NotImplementedError("write your pallas kernel here")


# speedup vs baseline: 3.2220x; 3.2220x over previous
import functools

import jax
import jax.numpy as jnp
from jax import lax
from jax.experimental import pallas as pl
from jax.experimental.pallas import tpu as pltpu
from jax.experimental.pallas import tpu_sc as plsc

N_CORES = 2
N_SUBCORES = 16
N_WORKERS = N_CORES * N_SUBCORES
EDGE_CHUNK = 200


def _mlp_body(x_ref, w1_ref, b1_ref, w2_ref, b2_ref, logit_ref, conf_ref):
    h = jnp.dot(x_ref[...], w1_ref[...], preferred_element_type=jnp.float32)
    h = jnp.maximum(h + b1_ref[...][None, :], 0.0)
    logits = jnp.dot(h, w2_ref[...], preferred_element_type=jnp.float32)
    logits = logits + b2_ref[...][None, :]
    m = jnp.max(logits, axis=1, keepdims=True)
    conf_ref[...] = 1.0 / jnp.sum(jnp.exp(logits - m), axis=1, keepdims=True)
    logit_ref[...] = logits


def _mlp_logits(x, W1, b1, W2, b2):
    n, d = x.shape
    hidden = W1.shape[1]
    n_proto = W2.shape[1]
    blk = 1000
    return pl.pallas_call(
        _mlp_body,
        grid=(n // blk,),
        in_specs=[
            pl.BlockSpec((blk, d), lambda i: (i, 0)),
            pl.BlockSpec((d, hidden), lambda i: (0, 0)),
            pl.BlockSpec((hidden,), lambda i: (0,)),
            pl.BlockSpec((hidden, n_proto), lambda i: (0, 0)),
            pl.BlockSpec((n_proto,), lambda i: (0,)),
        ],
        out_specs=[
            pl.BlockSpec((blk, n_proto), lambda i: (i, 0)),
            pl.BlockSpec((blk, 1), lambda i: (i, 0)),
        ],
        out_shape=[
            jax.ShapeDtypeStruct((n, n_proto), jnp.float32),
            jax.ShapeDtypeStruct((n, 1), jnp.float32),
        ],
    )(x, W1, b1, W2, b2)


def _sc_segment_sums(logits, sampled, row, col):
    n, width = logits.shape
    s_total = sampled.shape[0]
    per_w = s_total // N_WORKERS
    n_chunks = per_w // EDGE_CHUNK
    rows_per_tile = (-(-n // N_SUBCORES) + 7) // 8 * 8
    n_pad = rows_per_tile * N_SUBCORES
    mesh = plsc.VectorSubcoreMesh(core_axis_name="c", subcore_axis_name="s")

    @functools.partial(
        pl.kernel,
        mesh=mesh,
        out_type=jax.ShapeDtypeStruct((N_CORES, n_pad, width), jnp.float32),
        scratch_types=[
            pltpu.VMEM((EDGE_CHUNK,), jnp.int32),
            pltpu.VMEM((EDGE_CHUNK,), jnp.int32),
            pltpu.VMEM((EDGE_CHUNK,), jnp.int32),
            pltpu.VMEM((EDGE_CHUNK, width), jnp.float32),
            pltpu.VMEM_SHARED((n_pad, width), jnp.float32),
            pltpu.SemaphoreType.DMA,
        ],
    )
    def k(logit_hbm, samp_hbm, row_hbm, col_hbm, zero_hbm, out_hbm,
          samp_v, rowi_v, coli_v, rows_v, acc_sh, sem):
        c = lax.axis_index("c")
        s = lax.axis_index("s")
        wid = s * N_CORES + c
        pltpu.sync_copy(zero_hbm,
                        acc_sh.at[pl.ds(s * rows_per_tile, rows_per_tile)])
        plsc.subcore_barrier()
        for kk in range(n_chunks):
            base = wid * per_w + kk * EDGE_CHUNK
            pltpu.sync_copy(samp_hbm.at[pl.ds(base, EDGE_CHUNK)], samp_v)
            pltpu.async_copy(row_hbm.at[samp_v], rowi_v, sem).wait()
            pltpu.async_copy(col_hbm.at[samp_v], coli_v, sem).wait()
            pltpu.async_copy(logit_hbm.at[coli_v], rows_v, sem).wait()
            pltpu.sync_copy(rows_v, acc_sh.at[rowi_v], add=True)
        plsc.subcore_barrier()
        pltpu.sync_copy(
            acc_sh.at[pl.ds(s * rows_per_tile, rows_per_tile)],
            out_hbm.at[c, pl.ds(s * rows_per_tile, rows_per_tile)])

    zero = jnp.zeros((rows_per_tile, width), jnp.float32)
    return k(logits, sampled, row, col, zero)


def _combine_body(part_ref, lab_ref):
    s = part_ref[0] + part_ref[1]
    rowmax = jnp.max(s, axis=1, keepdims=True)
    idx = lax.broadcasted_iota(jnp.int32, s.shape, 1)
    first = jnp.min(jnp.where(s == rowmax, idx, s.shape[1]), axis=1)
    lab_ref[...] = first.astype(jnp.int32)[:, None]


def _combine(part, n):
    blk = 1000
    width = part.shape[2]
    return pl.pallas_call(
        _combine_body,
        grid=(n // blk,),
        in_specs=[pl.BlockSpec((N_CORES, blk, width), lambda i: (0, i, 0))],
        out_specs=pl.BlockSpec((blk, 1), lambda i: (i, 0)),
        out_shape=jax.ShapeDtypeStruct((n, 1), jnp.int32),
    )(part)


def kernel(x, edge_index, W1, b1, W2, b2):
    row, col = edge_index[0], edge_index[1]
    e = row.shape[0]
    if e > 100000:
        sampled = jax.random.randint(
            jax.random.key(42), (e // 5,), 0, e, dtype=jnp.int32)
    else:
        sampled = jnp.arange(e, dtype=jnp.int32)
    logits, conf = _mlp_logits(x, W1, b1, W2, b2)
    part = _sc_segment_sums(logits, sampled, row, col)
    labels = _combine(part, x.shape[0])
    return labels.reshape(-1), conf.reshape(-1)


# speedup vs baseline: 3.7272x; 1.1568x over previous
import functools

import jax
import jax.numpy as jnp
from jax import lax
from jax.experimental import pallas as pl
from jax.experimental.pallas import tpu as pltpu
from jax.experimental.pallas import tpu_sc as plsc

N_CORES = 2
N_SUBCORES = 16
N_WORKERS = N_CORES * N_SUBCORES
EDGE_CHUNK = 184


def _mlp_body(x_ref, w1_ref, b1_ref, w2_ref, b2_ref, logit_ref, conf_ref):
    h = jnp.dot(x_ref[...], w1_ref[...], preferred_element_type=jnp.float32)
    h = jnp.maximum(h + b1_ref[...][None, :], 0.0)
    logits = jnp.dot(h, w2_ref[...], preferred_element_type=jnp.float32)
    logits = logits + b2_ref[...][None, :]
    m = jnp.max(logits, axis=1, keepdims=True)
    conf_ref[...] = 1.0 / jnp.sum(jnp.exp(logits - m), axis=1, keepdims=True)
    logit_ref[...] = logits


def _mlp_logits(x, W1, b1, W2, b2):
    n, d = x.shape
    hidden = W1.shape[1]
    n_proto = W2.shape[1]
    blk = 1000
    return pl.pallas_call(
        _mlp_body,
        grid=(n // blk,),
        in_specs=[
            pl.BlockSpec((blk, d), lambda i: (i, 0)),
            pl.BlockSpec((d, hidden), lambda i: (0, 0)),
            pl.BlockSpec((hidden,), lambda i: (0,)),
            pl.BlockSpec((hidden, n_proto), lambda i: (0, 0)),
            pl.BlockSpec((n_proto,), lambda i: (0,)),
        ],
        out_specs=[
            pl.BlockSpec((blk, n_proto), lambda i: (i, 0)),
            pl.BlockSpec((blk, 1), lambda i: (i, 0)),
        ],
        out_shape=[
            jax.ShapeDtypeStruct((n, n_proto), jnp.float32),
            jax.ShapeDtypeStruct((n, 1), jnp.float32),
        ],
    )(x, W1, b1, W2, b2)


def _sc_segment_sums(logits, sampled, row, col):
    n, width = logits.shape
    s_total = sampled.shape[0]
    per_w = s_total // N_WORKERS
    n_full = per_w // EDGE_CHUNK
    rem = per_w - n_full * EDGE_CHUNK
    sizes = [EDGE_CHUNK] * n_full + ([rem] if rem else [])
    offs = [i * EDGE_CHUNK for i in range(len(sizes))]
    n_chunks = len(sizes)
    step = (n // N_SUBCORES) // 8 * 8
    win = n - (N_SUBCORES - 1) * step
    mesh = plsc.VectorSubcoreMesh(core_axis_name="c", subcore_axis_name="s")

    @functools.partial(
        pl.kernel,
        mesh=mesh,
        out_type=jax.ShapeDtypeStruct((N_CORES, n, width), jnp.float32),
        scratch_types=[
            pltpu.VMEM((per_w,), jnp.int32),
            [pltpu.VMEM((sz,), jnp.int32) for sz in sizes],
            pltpu.VMEM((per_w,), jnp.int32),
            [pltpu.VMEM((EDGE_CHUNK, width), jnp.float32) for _ in range(2)],
            pltpu.VMEM_SHARED((n, width), jnp.float32),
            pltpu.SemaphoreType.DMA,
            [pltpu.SemaphoreType.DMA for _ in range(2)],
            [pltpu.SemaphoreType.DMA for _ in range(2)],
        ],
    )
    def k(logit_hbm, samp_hbm, row_hbm, col_hbm, zero_hbm, out_hbm,
          samp_v, rowi_v, coli_v, rows_v, acc_sh, sem_z, sem_g, sem_s):
        c = lax.axis_index("c")
        s = lax.axis_index("s")
        wid = s * N_CORES + c
        zoff = jnp.minimum(s * step, n - win)
        zero_d = pltpu.async_copy(zero_hbm, acc_sh.at[pl.ds(zoff, win)],
                                  sem_z)
        pltpu.sync_copy(samp_hbm.at[pl.ds(wid * per_w, per_w)], samp_v)
        idx_d = [pltpu.async_copy(col_hbm.at[samp_v], coli_v, sem_z)]
        for kk in range(n_chunks):
            sl = samp_v.at[pl.ds(offs[kk], sizes[kk])]
            idx_d.append(pltpu.async_copy(row_hbm.at[sl], rowi_v[kk], sem_z))
        for d in idx_d:
            d.wait()
        zero_d.wait()
        plsc.subcore_barrier()
        def gather(kk, b):
            return pltpu.async_copy(
                logit_hbm.at[coli_v.at[pl.ds(offs[kk], sizes[kk])]],
                rows_v[b].at[pl.ds(0, sizes[kk])], sem_g[b])

        gat = [None, None]
        sca = [None, None]
        for kk in range(min(2, n_chunks)):
            gat[kk] = gather(kk, kk)
        for kk in range(n_chunks):
            b = kk % 2
            gat[b].wait()
            sca[b] = pltpu.async_copy(
                rows_v[b].at[pl.ds(0, sizes[kk])], acc_sh.at[rowi_v[kk]],
                sem_s[b], add=True)
            if kk + 2 < n_chunks:
                sca[b].wait()
                sca[b] = None
                gat[b] = gather(kk + 2, b)
        for d in sca:
            if d is not None:
                d.wait()
        plsc.subcore_barrier()
        pltpu.sync_copy(acc_sh.at[pl.ds(zoff, win)],
                        out_hbm.at[c, pl.ds(zoff, win)])

    zero = jnp.zeros((win, width), jnp.float32)
    return k(logits, sampled, row, col, zero)


def _combine_body(part_ref, lab_ref):
    s = part_ref[0] + part_ref[1]
    rowmax = jnp.max(s, axis=1, keepdims=True)
    idx = lax.broadcasted_iota(jnp.int32, s.shape, 1)
    first = jnp.min(jnp.where(s == rowmax, idx, s.shape[1]), axis=1)
    lab_ref[...] = first.astype(jnp.int32)[:, None]


def _combine(part, n):
    blk = 1000
    width = part.shape[2]
    return pl.pallas_call(
        _combine_body,
        grid=(n // blk,),
        in_specs=[pl.BlockSpec((N_CORES, blk, width), lambda i: (0, i, 0))],
        out_specs=pl.BlockSpec((blk, 1), lambda i: (i, 0)),
        out_shape=jax.ShapeDtypeStruct((n, 1), jnp.int32),
    )(part)


def kernel(x, edge_index, W1, b1, W2, b2):
    row, col = edge_index[0], edge_index[1]
    e = row.shape[0]
    if e > 100000:
        sampled = jax.random.randint(
            jax.random.key(42), (e // 5,), 0, e, dtype=jnp.int32)
    else:
        sampled = jnp.arange(e, dtype=jnp.int32)
    logits, conf = _mlp_logits(x, W1, b1, W2, b2)
    part = _sc_segment_sums(logits, sampled, row, col)
    labels = _combine(part, x.shape[0])
    return labels.reshape(-1), conf.reshape(-1)
